# baseline (speedup 1.0000x reference)
import functools

import jax
import jax.numpy as jnp
from jax import lax
from jax.experimental import pallas as pl
from jax.experimental.pallas import tpu as pltpu
from jax.experimental.pallas import tpu_sc as plsc

_B, _T, _D = 4, 2048, 128
_NC, _NS, _L = 2, 16, 16
_NW = _NC * _NS
_CHUNK = _T * _D // _NW
_SC_B = 1


def _sc_add(x_flat, pos_flat):
    mesh = plsc.VectorSubcoreMesh(core_axis_name="c", subcore_axis_name="s")

    @functools.partial(
        pl.kernel,
        out_type=jax.ShapeDtypeStruct((_SC_B * _T * _D,), jnp.float32),
        mesh=mesh,
        scratch_types=[
            pltpu.VMEM((_CHUNK,), jnp.float32),
            pltpu.VMEM((_SC_B, _CHUNK), jnp.float32),
            pltpu.SemaphoreType.DMA,
            pltpu.SemaphoreType.DMA,
        ],
    )
    def k(x_hbm, pos_hbm, out_hbm, pos_v, bufs, si, so):
        wid = lax.axis_index("s") * _NC + lax.axis_index("c")
        pbase = wid * _CHUNK
        loads = []
        for b in range(_SC_B):
            base = b * _T * _D + pbase
            loads.append(
                pltpu.async_copy(x_hbm.at[pl.ds(base, _CHUNK)],
                                 bufs.at[b], si))
        pltpu.sync_copy(pos_hbm.at[pl.ds(pbase, _CHUNK)], pos_v)
        stores = []
        for b in range(_SC_B):
            loads[b].wait()
            xb = bufs.at[b]

            def body(i, xb=xb):
                plsc.addupdate(xb.at[pl.ds(i, _L)],
                               pos_v.at[pl.ds(i, _L)][...])

            plsc.parallel_loop(0, _CHUNK, _L, unroll=8)(body)
            base = b * _T * _D + pbase
            stores.append(
                pltpu.async_copy(xb, out_hbm.at[pl.ds(base, _CHUNK)], so))
        for st in stores:
            st.wait()

    return k(x_flat, pos_flat)


def _tc_body(x_ref, p_ref, o_ref):
    o_ref[...] = x_ref[...] + p_ref[...]


def _tc_add(x_tail, pos_table):
    nb = _B - _SC_B
    return pl.pallas_call(
        _tc_body,
        grid=(nb,),
        in_specs=[
            pl.BlockSpec((1, _T, _D), lambda b: (b, 0, 0)),
            pl.BlockSpec((_T, _D), lambda b: (0, 0)),
        ],
        out_specs=pl.BlockSpec((1, _T, _D), lambda b: (b, 0, 0)),
        out_shape=jax.ShapeDtypeStruct((nb, _T, _D), jnp.float32),
    )(x_tail, pos_table)


@jax.jit
def _hybrid(x, pos_table):
    head = _sc_add(x[:_SC_B].reshape(-1), pos_table.reshape(-1))
    tail = _tc_add(x[_SC_B:], pos_table)
    return jnp.concatenate([head.reshape(_SC_B, _T, _D), tail], axis=0)


def kernel(x, pos_table):
    return _hybrid(x, pos_table)


# speedup vs baseline: 1.2303x; 1.2303x over previous
import functools

import jax
import jax.numpy as jnp
from jax import lax
from jax.experimental import pallas as pl
from jax.experimental.pallas import tpu as pltpu
from jax.experimental.pallas import tpu_sc as plsc

_B, _T, _D = 4, 2048, 128
_NC, _NS, _L = 2, 16, 16
_NW = _NC * _NS
_CHUNK = _T * _D // _NW


@jax.jit
def _sc_add(x_flat, pos_flat):
    mesh = plsc.VectorSubcoreMesh(core_axis_name="c", subcore_axis_name="s")

    @functools.partial(
        pl.kernel,
        out_type=jax.ShapeDtypeStruct((_B * _T * _D,), jnp.float32),
        mesh=mesh,
        scratch_types=[
            pltpu.VMEM((_CHUNK,), jnp.float32),
            pltpu.VMEM((_B, _CHUNK), jnp.float32),
            pltpu.SemaphoreType.DMA,
            pltpu.SemaphoreType.DMA,
            pltpu.SemaphoreType.DMA,
            pltpu.SemaphoreType.DMA,
            pltpu.SemaphoreType.DMA,
            pltpu.SemaphoreType.DMA,
        ],
    )
    def k(x_hbm, pos_hbm, out_hbm, pos_v, bufs, sp, s0, s1, s2, s3, so):
        isems = (s0, s1, s2, s3)
        wid = lax.axis_index("s") * _NC + lax.axis_index("c")
        pbase = wid * _CHUNK
        pload = pltpu.async_copy(pos_hbm.at[pl.ds(pbase, _CHUNK)], pos_v, sp)
        loads = []
        for b in range(_B):
            base = b * _T * _D + pbase
            loads.append(
                pltpu.async_copy(x_hbm.at[pl.ds(base, _CHUNK)],
                                 bufs.at[b], isems[b]))
        pload.wait()
        stores = []
        for b in range(_B):
            loads[b].wait()
            xb = bufs.at[b]

            def body(i, xb=xb):
                plsc.addupdate(xb.at[pl.ds(i, _L)],
                               pos_v.at[pl.ds(i, _L)][...])

            plsc.parallel_loop(0, _CHUNK, _L, unroll=16)(body)
            base = b * _T * _D + pbase
            stores.append(
                pltpu.async_copy(xb, out_hbm.at[pl.ds(base, _CHUNK)], so))
        for st in stores:
            st.wait()

    return k(x_flat, pos_flat)


def kernel(x, pos_table):
    out = _sc_add(x.reshape(-1), pos_table.reshape(-1))
    return out.reshape(_B, _T, _D)
